# Initial kernel scaffold; baseline (speedup 1.0000x reference)
import jax
import jax.numpy as jnp
from jax.experimental import pallas as pl


def kernel(features, coords, edge_index, kernel_offsets, kernel):
    raise NotImplementedError("write your pallas kernel here")


# speedup vs baseline: 3.8735x; 3.8735x over previous
import functools

import jax
import jax.numpy as jnp
from jax import lax
from jax.experimental import pallas as pl
from jax.experimental.pallas import tpu as pltpu
from jax.experimental.pallas import tpu_sc as plsc

N = 10000
E = 320000
INC = 128
OUTC = 128
K = 27

NC = 2
NS = 16
NW = NC * NS

EPW = E // NW
CHUNK = 80
NCHUNK = EPW // CHUNK
NP = 10240
RPT = NP // NS


def _matmul_body(f_ref, w_ref, o_ref):
    o_ref[0] = jnp.dot(f_ref[...], w_ref[0],
                       preferred_element_type=jnp.float32)


def _fx_all_offsets(features, kernel):
    return pl.pallas_call(
        _matmul_body,
        grid=(K,),
        in_specs=[
            pl.BlockSpec((N, INC), lambda k: (0, 0)),
            pl.BlockSpec((1, INC, OUTC), lambda k: (k, 0, 0)),
        ],
        out_specs=pl.BlockSpec((1, N, OUTC), lambda k: (k, 0, 0)),
        out_shape=jax.ShapeDtypeStruct((K, N, OUTC), jnp.float32),
    )(features, kernel)


_sc_mesh = plsc.VectorSubcoreMesh(core_axis_name="c", subcore_axis_name="s")


@functools.partial(
    pl.kernel,
    out_type=jax.ShapeDtypeStruct((NC, NP, OUTC), jnp.float32),
    mesh=_sc_mesh,
    scratch_types=[
        pltpu.VMEM((NCHUNK, CHUNK), jnp.int32),
        pltpu.VMEM((NCHUNK, CHUNK), jnp.int32),
        pltpu.VMEM((CHUNK, OUTC), jnp.float32),
        pltpu.VMEM_SHARED((NP, OUTC), jnp.float32),
        pltpu.SemaphoreType.DMA,
    ],
)
def _sc_gather_scatter(fx_hbm, gidx_hbm, didx_hbm, zro_hbm, out_hbm,
                       gidx_v, didx_v, rows_v, acc, sem):
    cid = lax.axis_index("c")
    sid = lax.axis_index("s")
    wid = cid * NS + sid

    pltpu.sync_copy(zro_hbm, acc.at[pl.ds(sid * RPT, RPT)])
    pltpu.sync_copy(gidx_hbm.at[wid], gidx_v)
    pltpu.sync_copy(didx_hbm.at[wid], didx_v)
    plsc.subcore_barrier()

    def chunk_body(j, carry):
        pltpu.async_copy(fx_hbm.at[gidx_v.at[j]], rows_v, sem).wait()
        pltpu.sync_copy(rows_v, acc.at[didx_v.at[j]], add=True)
        return carry

    lax.fori_loop(0, NCHUNK, chunk_body, 0)

    plsc.subcore_barrier()
    pltpu.sync_copy(acc.at[pl.ds(sid * RPT, RPT)],
                    out_hbm.at[cid, pl.ds(sid * RPT, RPT)])


def _sum_body(p_ref, o_ref):
    o_ref[...] = p_ref[0] + p_ref[1]


def _sum_partials(partials):
    bn = 2000
    return pl.pallas_call(
        _sum_body,
        grid=(N // bn,),
        in_specs=[pl.BlockSpec((NC, bn, OUTC), lambda i: (0, i, 0))],
        out_specs=pl.BlockSpec((bn, OUTC), lambda i: (i, 0)),
        out_shape=jax.ShapeDtypeStruct((N, OUTC), jnp.float32),
    )(partials)


@jax.jit
def kernel(features, coords, edge_index, kernel_offsets, kernel):
    del coords
    fx = _fx_all_offsets(features, kernel)
    fx2 = fx.reshape(K * N, OUTC)

    src = edge_index[0]
    dst = edge_index[1]
    gidx = (kernel_offsets * N + src).reshape(NW, NCHUNK, CHUNK)
    didx = dst.reshape(NW, NCHUNK, CHUNK)
    zro = jnp.zeros((RPT, OUTC), jnp.float32)

    partials = _sc_gather_scatter(fx2, gidx, didx, zro)
    return _sum_partials(partials)


# speedup vs baseline: 5.1061x; 1.3182x over previous
import functools

import jax
import jax.numpy as jnp
from jax import lax
from jax.experimental import pallas as pl
from jax.experimental.pallas import tpu as pltpu
from jax.experimental.pallas import tpu_sc as plsc

N = 10000
E = 320000
INC = 128
OUTC = 128
K = 27

NC = 2
NS = 16
NW = NC * NS

EPW = E // NW
CHUNK = 40
NCHUNK = EPW // CHUNK
NP = 10240
RPT = NP // NS


def _matmul_body(f_ref, w_ref, o_ref):
    o_ref[0] = jnp.dot(f_ref[...], w_ref[0],
                       preferred_element_type=jnp.float32)


def _fx_all_offsets(features, kernel):
    return pl.pallas_call(
        _matmul_body,
        grid=(K,),
        in_specs=[
            pl.BlockSpec((N, INC), lambda k: (0, 0)),
            pl.BlockSpec((1, INC, OUTC), lambda k: (k, 0, 0)),
        ],
        out_specs=pl.BlockSpec((1, N, OUTC), lambda k: (k, 0, 0)),
        out_shape=jax.ShapeDtypeStruct((K, N, OUTC), jnp.float32),
    )(features, kernel)


_sc_mesh = plsc.VectorSubcoreMesh(core_axis_name="c", subcore_axis_name="s")


NBUF = 5


@functools.partial(
    pl.kernel,
    out_type=jax.ShapeDtypeStruct((NC, NP, OUTC), jnp.float32),
    mesh=_sc_mesh,
    scratch_types=(
        [pltpu.VMEM((CHUNK,), jnp.int32) for _ in range(NBUF)]
        + [pltpu.VMEM((CHUNK,), jnp.int32) for _ in range(NBUF)]
        + [pltpu.VMEM((CHUNK, OUTC), jnp.float32) for _ in range(NBUF)]
        + [pltpu.VMEM_SHARED((NP, OUTC), jnp.float32)]
        + [pltpu.SemaphoreType.DMA for _ in range(3 * NBUF)]
    ),
)
def _sc_gather_scatter(fx_hbm, gidx_hbm, didx_hbm, zro_hbm, out_hbm, *rest):
    idxg = rest[:NBUF]
    idxd = rest[NBUF:2 * NBUF]
    rows = rest[2 * NBUF:3 * NBUF]
    acc = rest[3 * NBUF]
    isem = rest[3 * NBUF + 1:4 * NBUF + 1]
    gsem = rest[4 * NBUF + 1:5 * NBUF + 1]
    ssem = rest[5 * NBUF + 1:]

    cid = lax.axis_index("c")
    sid = lax.axis_index("s")
    wid = cid * NS + sid
    base = wid * EPW

    pltpu.sync_copy(zro_hbm, acc.at[pl.ds(sid * RPT, RPT)])
    plsc.subcore_barrier()

    def fire_idx(j, b):
        off = base + j * CHUNK
        pltpu.async_copy(gidx_hbm.at[pl.ds(off, CHUNK)], idxg[b], isem[b])
        pltpu.async_copy(didx_hbm.at[pl.ds(off, CHUNK)], idxd[b], isem[b])

    def wait_idx(b):
        pltpu.make_async_copy(gidx_hbm.at[pl.ds(0, CHUNK)], idxg[b],
                              isem[b]).wait()
        pltpu.make_async_copy(didx_hbm.at[pl.ds(0, CHUNK)], idxd[b],
                              isem[b]).wait()

    def fire_gather(b):
        pltpu.async_copy(fx_hbm.at[idxg[b]], rows[b], gsem[b])

    def wait_gather(b):
        pltpu.make_async_copy(fx_hbm.at[idxg[b]], rows[b], gsem[b]).wait()

    def fire_scatter(b):
        pltpu.async_copy(rows[b], acc.at[idxd[b]], ssem[b], add=True)

    def wait_scatter(b):
        pltpu.make_async_copy(rows[b], acc.at[idxd[b]], ssem[b]).wait()

    for b in range(NBUF):
        fire_idx(b, b)
    for b in range(NBUF):
        wait_idx(b)
        fire_gather(b)

    def group_body(g, carry):
        for b in range(NBUF):
            wait_gather(b)
            fire_scatter(b)
        for b in range(NBUF):
            wait_scatter(b)
            fire_idx((g + 1) * NBUF + b, b)
        for b in range(NBUF):
            wait_idx(b)
            fire_gather(b)
        return carry

    lax.fori_loop(0, NCHUNK // NBUF - 1, group_body, 0)

    for b in range(NBUF):
        wait_gather(b)
        fire_scatter(b)
    for b in range(NBUF):
        wait_scatter(b)

    plsc.subcore_barrier()
    pltpu.sync_copy(acc.at[pl.ds(sid * RPT, RPT)],
                    out_hbm.at[cid, pl.ds(sid * RPT, RPT)])


def _sum_body(p_ref, o_ref):
    o_ref[...] = p_ref[0] + p_ref[1]


def _sum_partials(partials):
    bn = 2000
    return pl.pallas_call(
        _sum_body,
        grid=(N // bn,),
        in_specs=[pl.BlockSpec((NC, bn, OUTC), lambda i: (0, i, 0))],
        out_specs=pl.BlockSpec((bn, OUTC), lambda i: (i, 0)),
        out_shape=jax.ShapeDtypeStruct((N, OUTC), jnp.float32),
    )(partials)


@jax.jit
def kernel(features, coords, edge_index, kernel_offsets, kernel):
    del coords
    fx = _fx_all_offsets(features, kernel)
    fx2 = fx.reshape(K * N, OUTC)

    src = edge_index[0]
    dst = edge_index[1]
    gidx = kernel_offsets * N + src
    didx = dst
    zro = jnp.zeros((RPT, OUTC), jnp.float32)

    partials = _sc_gather_scatter(fx2, gidx, didx, zro)
    return _sum_partials(partials)
